# baseline (speedup 1.0000x reference)
import jax
import jax.numpy as jnp
from jax.experimental import pallas as pl
from jax.experimental.pallas import tpu as pltpu

_BLK = 8


def _scale_body(xr_ref, xi_ref, br_ref, bi_ref, or_ref, oi_ref):
    sr = jnp.exp(br_ref[...])
    si = jnp.exp(bi_ref[...])
    or_ref[...] = xr_ref[...] * sr[None, :, :]
    oi_ref[...] = xi_ref[...] * si[None, :, :]


def kernel(x_real, x_imag, betas_real, betas_imag):
    b, c, h, w = x_real.shape
    n = b * c
    xr = x_real.reshape(n, h, w)
    xi = x_imag.reshape(n, h, w)
    br = betas_real.reshape(h, w)
    bi = betas_imag.reshape(h, w)

    blk = _BLK if n % _BLK == 0 else 1
    grid = (n // blk,)

    x_spec = pl.BlockSpec((blk, h, w), lambda i: (i, 0, 0))
    b_spec = pl.BlockSpec((h, w), lambda i: (0, 0))

    out_r, out_i = pl.pallas_call(
        _scale_body,
        grid=grid,
        in_specs=[x_spec, x_spec, b_spec, b_spec],
        out_specs=[x_spec, x_spec],
        out_shape=[
            jax.ShapeDtypeStruct((n, h, w), x_real.dtype),
            jax.ShapeDtypeStruct((n, h, w), x_imag.dtype),
        ],
        compiler_params=pltpu.CompilerParams(
            dimension_semantics=("parallel",),
        ),
    )(xr, xi, br, bi)

    return (out_r.reshape(b, c, h, w), out_i.reshape(b, c, h, w))


# speedup vs baseline: 1.0122x; 1.0122x over previous
import jax
import jax.numpy as jnp
from jax.experimental import pallas as pl
from jax.experimental.pallas import tpu as pltpu

_BLK = 16


def _scale_body(xr_ref, xi_ref, br_ref, bi_ref, or_ref, oi_ref):
    sr = jnp.exp(br_ref[...])
    si = jnp.exp(bi_ref[...])
    or_ref[...] = xr_ref[...] * sr[None, :, :]
    oi_ref[...] = xi_ref[...] * si[None, :, :]


def kernel(x_real, x_imag, betas_real, betas_imag):
    b, c, h, w = x_real.shape
    n = b * c
    xr = x_real.reshape(n, h, w)
    xi = x_imag.reshape(n, h, w)
    br = betas_real.reshape(h, w)
    bi = betas_imag.reshape(h, w)

    blk = _BLK if n % _BLK == 0 else 1
    grid = (n // blk,)

    x_spec = pl.BlockSpec((blk, h, w), lambda i: (i, 0, 0))
    b_spec = pl.BlockSpec((h, w), lambda i: (0, 0))

    out_r, out_i = pl.pallas_call(
        _scale_body,
        grid=grid,
        in_specs=[x_spec, x_spec, b_spec, b_spec],
        out_specs=[x_spec, x_spec],
        out_shape=[
            jax.ShapeDtypeStruct((n, h, w), x_real.dtype),
            jax.ShapeDtypeStruct((n, h, w), x_imag.dtype),
        ],
        compiler_params=pltpu.CompilerParams(
            dimension_semantics=("parallel",),
        ),
    )(xr, xi, br, bi)

    return (out_r.reshape(b, c, h, w), out_i.reshape(b, c, h, w))


# speedup vs baseline: 1.0135x; 1.0013x over previous
import jax
import jax.numpy as jnp
from jax.experimental import pallas as pl
from jax.experimental.pallas import tpu as pltpu

_BLK = 32


def _scale_body(x_ref, b_ref, o_ref):
    o_ref[...] = x_ref[...] * jnp.exp(b_ref[...])[None, :, :]


def _scale(x, betas):
    b, c, h, w = x.shape
    n = b * c
    xf = x.reshape(n, h, w)
    bf = betas.reshape(h, w)
    blk = _BLK if n % _BLK == 0 else 1
    out = pl.pallas_call(
        _scale_body,
        grid=(n // blk,),
        in_specs=[
            pl.BlockSpec((blk, h, w), lambda i: (i, 0, 0)),
            pl.BlockSpec((h, w), lambda i: (0, 0)),
        ],
        out_specs=pl.BlockSpec((blk, h, w), lambda i: (i, 0, 0)),
        out_shape=jax.ShapeDtypeStruct((n, h, w), x.dtype),
        compiler_params=pltpu.CompilerParams(
            dimension_semantics=("parallel",),
        ),
    )(xf, bf)
    return out.reshape(b, c, h, w)


def kernel(x_real, x_imag, betas_real, betas_imag):
    return (_scale(x_real, betas_real), _scale(x_imag, betas_imag))
